# baseline (speedup 1.0000x reference)
import functools

import jax
import jax.numpy as jnp
from jax import lax
from jax.experimental import pallas as pl
from jax.experimental.pallas import tpu as pltpu
from jax.experimental.pallas import tpu_sc as plsc

N = 10000
E = 320000
XD = 128
HD = 128
ZD = 64
T = 3

NC = 2
NS = 16
L = 16

NP = 10240
NR = NP // 128
EB = 80
NW = NC * NS
NBLK = 3 * -(-E // (NW * EB * 3))
EPW = NBLK * EB
EP = EPW * NW
CH = NP // NS

_mesh = plsc.VectorSubcoreMesh(
    core_axis_name="c", subcore_axis_name="s", num_cores=NC, num_subcores=NS)


@functools.partial(
    pl.kernel,
    out_type=jax.ShapeDtypeStruct((NC, NP), jnp.float32),
    mesh=_mesh,
    compiler_params=pltpu.CompilerParams(needs_layout_passes=False),
    scratch_types=[
        pltpu.VMEM((EPW,), jnp.int32),
        pltpu.VMEM((EPW,), jnp.float32),
        pltpu.VMEM((NP,), jnp.float32),
        pltpu.VMEM((NS, CH), jnp.float32),
        pltpu.VMEM((CH,), jnp.float32),
        pltpu.VMEM_SHARED((NS, NP), jnp.float32),
    ],
)
def _deg_kernel(dst_hbm, w_hbm, out_hbm, didx_v, w_v, deg_v, red_v, col_v,
                deg_sh):
    c = lax.axis_index("c")
    s = lax.axis_index("s")
    wid = c * NS + s
    zero16 = jnp.zeros((L,), jnp.float32)
    base = wid * EPW
    pltpu.sync_copy(dst_hbm.at[pl.ds(base, EPW)], didx_v)
    pltpu.sync_copy(w_hbm.at[pl.ds(base, EPW)], w_v)

    def zbody(i, _):
        deg_v[pl.ds(i * L, L)] = zero16
        return 0
    lax.fori_loop(0, NP // L, zbody, 0)

    def inner(j, _):
        idx = didx_v[pl.ds(j * L, L)]
        val = w_v[pl.ds(j * L, L)]
        plsc.addupdate_scatter(deg_v, [idx], val)
        return 0
    lax.fori_loop(0, EPW // L, inner, 0)

    pltpu.sync_copy(deg_v, deg_sh.at[s])
    plsc.subcore_barrier()
    pltpu.sync_copy(deg_sh.at[:, pl.ds(s * CH, CH)], red_v)

    def rbody(j, _):
        acc = red_v[0, pl.ds(j * L, L)]
        for r in range(1, NS):
            acc = acc + red_v[r, pl.ds(j * L, L)]
        col_v[pl.ds(j * L, L)] = acc
        return 0
    lax.fori_loop(0, CH // L, rbody, 0)
    pltpu.sync_copy(col_v, out_hbm.at[c, pl.ds(s * CH, CH)])


@functools.partial(
    pl.kernel,
    out_type=jax.ShapeDtypeStruct((NC, NP, HD), jnp.float32),
    mesh=_mesh,
    compiler_params=pltpu.CompilerParams(needs_layout_passes=False),
    scratch_types=[
        pltpu.VMEM((NBLK, EB), jnp.int32),
        pltpu.VMEM((EB,), jnp.int32),
        pltpu.VMEM((EB,), jnp.int32),
        pltpu.VMEM((EB,), jnp.int32),
        pltpu.VMEM((EB,), jnp.float32),
        pltpu.VMEM((EB,), jnp.float32),
        pltpu.VMEM((EB,), jnp.float32),
        pltpu.VMEM((EB, HD), jnp.float32),
        pltpu.VMEM((EB, HD), jnp.float32),
        pltpu.VMEM((EB, HD), jnp.float32),
        pltpu.SemaphoreType.DMA,
        pltpu.SemaphoreType.DMA,
        pltpu.SemaphoreType.DMA,
        pltpu.SemaphoreType.DMA,
        pltpu.SemaphoreType.DMA,
        pltpu.SemaphoreType.DMA,
        pltpu.SemaphoreType.DMA,
        pltpu.SemaphoreType.DMA,
        pltpu.SemaphoreType.DMA,
        pltpu.SemaphoreType.DMA,
        pltpu.SemaphoreType.DMA,
        pltpu.SemaphoreType.DMA,
        pltpu.VMEM_SHARED((NP, HD), jnp.float32),
    ],
)
def _agg_kernel(table_hbm, src_hbm, dst_hbm, w_hbm, out_hbm,
                sidx_v, didx0_v, didx1_v, didx2_v, w0_v, w1_v, w2_v,
                rows0_v, rows1_v, rows2_v,
                gs0, gs1, gs2, ss0, ss1, ss2, ds0, ds1, ds2, ws0, ws1, ws2,
                agg_sh):
    c = lax.axis_index("c")
    s = lax.axis_index("s")
    wid = c * NS + s
    zero16 = jnp.zeros((L,), jnp.float32)
    rows = (rows0_v, rows1_v, rows2_v)
    didxs = (didx0_v, didx1_v, didx2_v)
    ws = (w0_v, w1_v, w2_v)
    gsems = (gs0, gs1, gs2)
    ssems = (ss0, ss1, ss2)
    dsems = (ds0, ds1, ds2)
    wsems = (ws0, ws1, ws2)

    def zbody(i, _):
        rows0_v[i // (HD // L), pl.ds((i % (HD // L)) * L, L)] = zero16
        return 0
    lax.fori_loop(0, EB * HD // L, zbody, 0)
    for k in range(-(-CH // EB)):
        nrow = min(EB, CH - k * EB)
        pltpu.sync_copy(rows0_v.at[pl.ds(0, nrow)],
                        agg_sh.at[pl.ds(s * CH + k * EB, nrow)])
    pltpu.sync_copy(src_hbm.at[wid], sidx_v)
    plsc.subcore_barrier()

    def scale(p):
        def mbody(j, _):
            a16 = ws[p][pl.ds(j * L, L)]
            for k in range(L):
                cf = a16[k]
                e = j * L + k
                for f in range(HD // L):
                    sl = pl.ds(f * L, L)
                    rows[p][e, sl] = rows[p][e, sl] * cf
            return 0
        lax.fori_loop(0, EB // L, mbody, 0)

    def load_idx(b, p):
        pltpu.async_copy(dst_hbm.at[wid * NBLK + b], didxs[p], dsems[p])
        pltpu.async_copy(w_hbm.at[wid * NBLK + b], ws[p], wsems[p])

    def wait_idx(b, p):
        pltpu.make_async_copy(dst_hbm.at[wid * NBLK + b], didxs[p],
                              dsems[p]).wait()
        pltpu.make_async_copy(w_hbm.at[wid * NBLK + b], ws[p],
                              wsems[p]).wait()

    def gather(b, p):
        pltpu.async_copy(table_hbm.at[sidx_v.at[b]], rows[p], gsems[p])

    def wait_gather(b, p):
        pltpu.make_async_copy(table_hbm.at[sidx_v.at[b]], rows[p],
                              gsems[p]).wait()

    def scatter(p):
        pltpu.async_copy(rows[p], agg_sh.at[didxs[p]], ssems[p], add=True)

    def wait_scatter(p):
        pltpu.make_async_copy(rows[p], agg_sh.at[didxs[p]], ssems[p]).wait()

    load_idx(0, 0)
    load_idx(1, 1)
    gather(0, 0)
    gather(1, 1)

    def steady(b, p):
        wait_idx(b, p)
        wait_gather(b, p)
        scale(p)
        scatter(p)
        pn = (p + 2) % 3
        wait_scatter(pn)
        gather(b + 2, pn)
        load_idx(b + 2, pn)

    wait_idx(0, 0)
    wait_gather(0, 0)
    scale(0)
    scatter(0)
    gather(2, 2)
    load_idx(2, 2)

    def ebody(i, _):
        b = 1 + i * 3
        steady(b, 1)
        steady(b + 1, 2)
        steady(b + 2, 0)
        return 0
    lax.fori_loop(0, (NBLK - 3) // 3, ebody, 0)

    for b in (NBLK - 2, NBLK - 1):
        p = b % 3
        wait_idx(b, p)
        wait_gather(b, p)
        scale(p)
        scatter(p)
        wait_scatter((p + 2) % 3)
    wait_scatter((NBLK - 1) % 3)

    plsc.subcore_barrier()
    pltpu.sync_copy(agg_sh.at[pl.ds(s * CH, CH)],
                    out_hbm.at[c, pl.ds(s * CH, CH)])


RB = 1280


def _dinv_body(degp_ref, x_ref, w1_ref, table_ref, dinvb_ref):
    d = degp_ref[0] + degp_ref[1] + 1.0
    dv = lax.rsqrt(d)
    xw1 = jnp.dot(x_ref[...], w1_ref[...], preferred_element_type=jnp.float32)
    dt = dv.T
    for r in range(NR):
        blk = jnp.broadcast_to(dt[:, r:r + 1], (128, HD))
        sl = pl.ds(r * 128, 128)
        dinvb_ref[sl, :] = blk
        table_ref[sl, :] = blk * xw1[r * 128:(r + 1) * 128, :]


_dinv_tc = pl.pallas_call(
    _dinv_body,
    in_specs=[pl.BlockSpec((NC, NR, 128), lambda: (0, 0, 0)),
              pl.BlockSpec((NP, XD), lambda: (0, 0)),
              pl.BlockSpec((XD, HD), lambda: (0, 0))],
    out_specs=[pl.BlockSpec((NP, HD), lambda: (0, 0)),
               pl.BlockSpec((NP, HD), lambda: (0, 0))],
    out_shape=[jax.ShapeDtypeStruct((NP, HD), jnp.float32),
               jax.ShapeDtypeStruct((NP, HD), jnp.float32)],
    grid=(),
)


def _layer2_body(aggp_ref, t1_ref, dinvb_ref, b1_ref, w2_ref, o_ref):
    h = jax.nn.relu(dinvb_ref[...]
                    * (aggp_ref[0] + aggp_ref[1] + t1_ref[...])
                    + b1_ref[...])
    o_ref[...] = dinvb_ref[...] * jnp.dot(h, w2_ref[...],
                                          preferred_element_type=jnp.float32)


_layer2 = pl.pallas_call(
    _layer2_body,
    grid=(NP // RB,),
    in_specs=[pl.BlockSpec((NC, RB, HD), lambda i: (0, i, 0)),
              pl.BlockSpec((RB, HD), lambda i: (i, 0)),
              pl.BlockSpec((RB, HD), lambda i: (i, 0)),
              pl.BlockSpec((1, HD), lambda i: (0, 0)),
              pl.BlockSpec((HD, HD), lambda i: (0, 0))],
    out_specs=pl.BlockSpec((RB, HD), lambda i: (i, 0)),
    out_shape=jax.ShapeDtypeStruct((NP, HD), jnp.float32),
)


def _gru_body(aggp_ref, t2_ref, dinvb_ref, b2_ref, wz_ref, uz_ref, bz_ref,
              wr_ref, ur_ref, br_ref, wh_ref, uh_ref, bh_ref, wl_ref, bl_ref,
              out_ref, h_ref):
    def mm(a, b):
        return jnp.dot(a, b[...], preferred_element_type=jnp.float32)

    z = jnp.tanh(dinvb_ref[...] * (aggp_ref[0] + aggp_ref[1] + t2_ref[...])
                 + b2_ref[...])
    xz = mm(z, wz_ref) + bz_ref[...]
    xr = mm(z, wr_ref) + br_ref[...]
    xh = mm(z, wh_ref) + bh_ref[...]

    h1 = jax.nn.sigmoid(xz) * jnp.tanh(xh)
    zg = jax.nn.sigmoid(xz + mm(h1, uz_ref))
    rg = jax.nn.sigmoid(xr + mm(h1, ur_ref))
    hh = jnp.tanh(xh + mm(rg * h1, uh_ref))
    h2 = (1.0 - zg) * h1 + zg * hh
    zg = jax.nn.sigmoid(xz + mm(h2, uz_ref))
    rg = jax.nn.sigmoid(xr + mm(h2, ur_ref))
    hh = jnp.tanh(xh + mm(rg * h2, uh_ref))
    h3 = (1.0 - zg) * h2 + zg * hh

    bl = bl_ref[...]
    out_ref[0] = mm(h1, wl_ref) + bl
    out_ref[1] = mm(h2, wl_ref) + bl
    out_ref[2] = mm(h3, wl_ref) + bl
    h_ref[...] = h3


RB2 = 1000

_gru_tc = pl.pallas_call(
    _gru_body,
    grid=(N // RB2,),
    in_specs=[pl.BlockSpec((NC, RB2, HD), lambda i: (0, i, 0)),
              pl.BlockSpec((RB2, HD), lambda i: (i, 0)),
              pl.BlockSpec((RB2, HD), lambda i: (i, 0)),
              pl.BlockSpec((1, HD), lambda i: (0, 0)),
              pl.BlockSpec((HD, HD), lambda i: (0, 0)),
              pl.BlockSpec((HD, HD), lambda i: (0, 0)),
              pl.BlockSpec((1, HD), lambda i: (0, 0)),
              pl.BlockSpec((HD, HD), lambda i: (0, 0)),
              pl.BlockSpec((HD, HD), lambda i: (0, 0)),
              pl.BlockSpec((1, HD), lambda i: (0, 0)),
              pl.BlockSpec((HD, HD), lambda i: (0, 0)),
              pl.BlockSpec((HD, HD), lambda i: (0, 0)),
              pl.BlockSpec((1, HD), lambda i: (0, 0)),
              pl.BlockSpec((HD, ZD), lambda i: (0, 0)),
              pl.BlockSpec((1, ZD), lambda i: (0, 0))],
    out_specs=[pl.BlockSpec((T, RB2, ZD), lambda i: (0, i, 0)),
               pl.BlockSpec((RB2, HD), lambda i: (i, 0))],
    out_shape=[jax.ShapeDtypeStruct((T, N, ZD), jnp.float32),
               jax.ShapeDtypeStruct((N, HD), jnp.float32)],
)


def kernel(x, edge_index, edge_weight, W1, b1, W2, b2, Wz, Uz, bz,
           Wr, Ur, br, Wh, Uh, bh, Wl, bl):
    pad = EP - E
    src = jnp.pad(edge_index[0], (0, pad))
    dst = jnp.pad(edge_index[1], (0, pad))
    w = jnp.pad(edge_weight, (0, pad))
    src3 = src.reshape(NW, NBLK, EB)
    dst3 = dst.reshape(NW * NBLK, EB)
    w3 = w.reshape(NW * NBLK, EB)

    xp = jnp.pad(x, ((0, NP - N), (0, 0)))

    degp = _deg_kernel(dst, w)
    table1, dinvb = _dinv_tc(degp.reshape(NC, NR, 128), xp, W1)

    agg1 = _agg_kernel(table1, src3, dst3, w3)
    table2 = _layer2(agg1, table1, dinvb, b1.reshape(1, HD), W2)
    agg2 = _agg_kernel(table2, src3, dst3, w3)

    out, h = _gru_tc(agg2, table2, dinvb, b2.reshape(1, HD), Wz, Uz,
                     bz.reshape(1, HD), Wr, Ur, br.reshape(1, HD),
                     Wh, Uh, bh.reshape(1, HD), Wl, bl.reshape(1, ZD))
    return out, h


# speedup vs baseline: 1.7423x; 1.7423x over previous
import functools

import jax
import jax.numpy as jnp
from jax import lax
from jax.experimental import pallas as pl
from jax.experimental.pallas import tpu as pltpu
from jax.experimental.pallas import tpu_sc as plsc

N = 10000
E = 320000
XD = 128
HD = 128
ZD = 64
T = 3

NC = 2
NS = 16
L = 16

NP = 10240
NR = NP // 128
EB = 80
NW = NC * NS
NBLK = 3 * -(-E // (NW * EB * 3))
EPW = NBLK * EB
EP = EPW * NW
CH = NP // NS

_mesh = plsc.VectorSubcoreMesh(
    core_axis_name="c", subcore_axis_name="s", num_cores=NC, num_subcores=NS)


@functools.partial(
    pl.kernel,
    out_type=jax.ShapeDtypeStruct((NC, NP), jnp.float32),
    mesh=_mesh,
    compiler_params=pltpu.CompilerParams(needs_layout_passes=False),
    scratch_types=[
        pltpu.VMEM((EPW,), jnp.int32),
        pltpu.VMEM((EPW,), jnp.float32),
        pltpu.VMEM((NP,), jnp.float32),
        pltpu.VMEM((NS, CH), jnp.float32),
        pltpu.VMEM((CH,), jnp.float32),
        pltpu.VMEM_SHARED((NS, NP), jnp.float32),
    ],
)
def _deg_kernel(dst_hbm, w_hbm, out_hbm, didx_v, w_v, deg_v, red_v, col_v,
                deg_sh):
    c = lax.axis_index("c")
    s = lax.axis_index("s")
    wid = c * NS + s
    zero16 = jnp.zeros((L,), jnp.float32)
    base = wid * EPW
    pltpu.sync_copy(dst_hbm.at[pl.ds(base, EPW)], didx_v)
    pltpu.sync_copy(w_hbm.at[pl.ds(base, EPW)], w_v)

    def zbody(i, _):
        deg_v[pl.ds(i * L, L)] = zero16
        return 0
    lax.fori_loop(0, NP // L, zbody, 0)

    def inner(j, _):
        idx = didx_v[pl.ds(j * L, L)]
        val = w_v[pl.ds(j * L, L)]
        plsc.addupdate_scatter(deg_v, [idx], val)
        return 0
    lax.fori_loop(0, EPW // L, inner, 0)

    pltpu.sync_copy(deg_v, deg_sh.at[s])
    plsc.subcore_barrier()
    pltpu.sync_copy(deg_sh.at[:, pl.ds(s * CH, CH)], red_v)

    def rbody(j, _):
        acc = red_v[0, pl.ds(j * L, L)]
        for r in range(1, NS):
            acc = acc + red_v[r, pl.ds(j * L, L)]
        col_v[pl.ds(j * L, L)] = acc
        return 0
    lax.fori_loop(0, CH // L, rbody, 0)
    pltpu.sync_copy(col_v, out_hbm.at[c, pl.ds(s * CH, CH)])


@functools.partial(
    pl.kernel,
    out_type=jax.ShapeDtypeStruct((NC, NP, HD), jnp.float32),
    mesh=_mesh,
    compiler_params=pltpu.CompilerParams(needs_layout_passes=False),
    scratch_types=[
        pltpu.VMEM((NBLK, EB), jnp.int32),
        pltpu.VMEM((EB,), jnp.int32),
        pltpu.VMEM((EB,), jnp.int32),
        pltpu.VMEM((EB,), jnp.int32),
        pltpu.VMEM((EB,), jnp.float32),
        pltpu.VMEM((EB,), jnp.float32),
        pltpu.VMEM((EB,), jnp.float32),
        pltpu.VMEM((EB, HD), jnp.float32),
        pltpu.VMEM((EB, HD), jnp.float32),
        pltpu.VMEM((EB, HD), jnp.float32),
        pltpu.SemaphoreType.DMA,
        pltpu.SemaphoreType.DMA,
        pltpu.SemaphoreType.DMA,
        pltpu.SemaphoreType.DMA,
        pltpu.SemaphoreType.DMA,
        pltpu.SemaphoreType.DMA,
        pltpu.SemaphoreType.DMA,
        pltpu.SemaphoreType.DMA,
        pltpu.SemaphoreType.DMA,
        pltpu.SemaphoreType.DMA,
        pltpu.SemaphoreType.DMA,
        pltpu.SemaphoreType.DMA,
        pltpu.VMEM_SHARED((NP, HD), jnp.float32),
    ],
)
def _agg_kernel(table_hbm, src_hbm, dst_hbm, w_hbm, out_hbm,
                sidx_v, didx0_v, didx1_v, didx2_v, w0_v, w1_v, w2_v,
                rows0_v, rows1_v, rows2_v,
                gs0, gs1, gs2, ss0, ss1, ss2, ds0, ds1, ds2, ws0, ws1, ws2,
                agg_sh):
    c = lax.axis_index("c")
    s = lax.axis_index("s")
    wid = c * NS + s
    zero16 = jnp.zeros((L,), jnp.float32)
    rows = (rows0_v, rows1_v, rows2_v)
    didxs = (didx0_v, didx1_v, didx2_v)
    ws = (w0_v, w1_v, w2_v)
    gsems = (gs0, gs1, gs2)
    ssems = (ss0, ss1, ss2)
    dsems = (ds0, ds1, ds2)
    wsems = (ws0, ws1, ws2)

    def zbody(i, _):
        rows0_v[i // (HD // L), pl.ds((i % (HD // L)) * L, L)] = zero16
        return 0
    lax.fori_loop(0, EB * HD // L, zbody, 0)
    for k in range(-(-CH // EB)):
        nrow = min(EB, CH - k * EB)
        pltpu.sync_copy(rows0_v.at[pl.ds(0, nrow)],
                        agg_sh.at[pl.ds(s * CH + k * EB, nrow)])
    pltpu.sync_copy(src_hbm.at[wid], sidx_v)
    plsc.subcore_barrier()

    def scale(p):
        def mbody(j, _):
            a16 = ws[p][pl.ds(j * L, L)]
            for k in range(L):
                cf = a16[k]
                e = j * L + k
                for f in range(HD // L):
                    sl = pl.ds(f * L, L)
                    rows[p][e, sl] = rows[p][e, sl] * cf
            return 0
        lax.fori_loop(0, EB // L, mbody, 0)

    def load_idx(b, p):
        pltpu.async_copy(dst_hbm.at[wid * NBLK + b], didxs[p], dsems[p])
        pltpu.async_copy(w_hbm.at[wid * NBLK + b], ws[p], wsems[p])

    def wait_idx(b, p):
        pltpu.make_async_copy(dst_hbm.at[wid * NBLK + b], didxs[p],
                              dsems[p]).wait()
        pltpu.make_async_copy(w_hbm.at[wid * NBLK + b], ws[p],
                              wsems[p]).wait()

    def gather(b, p):
        pltpu.async_copy(table_hbm.at[sidx_v.at[b]], rows[p], gsems[p])

    def wait_gather(b, p):
        pltpu.make_async_copy(table_hbm.at[sidx_v.at[b]], rows[p],
                              gsems[p]).wait()

    def scatter(p):
        pltpu.async_copy(rows[p], agg_sh.at[didxs[p]], ssems[p], add=True)

    def wait_scatter(p):
        pltpu.make_async_copy(rows[p], agg_sh.at[didxs[p]], ssems[p]).wait()

    load_idx(0, 0)
    load_idx(1, 1)
    gather(0, 0)
    gather(1, 1)

    def steady(b, p):
        wait_idx(b, p)
        wait_gather(b, p)
        scale(p)
        scatter(p)
        pn = (p + 2) % 3
        wait_scatter(pn)
        gather(b + 2, pn)
        load_idx(b + 2, pn)

    wait_idx(0, 0)
    wait_gather(0, 0)
    scale(0)
    scatter(0)
    gather(2, 2)
    load_idx(2, 2)

    def ebody(i, _):
        b = 1 + i * 3
        steady(b, 1)
        steady(b + 1, 2)
        steady(b + 2, 0)
        return 0
    lax.fori_loop(0, (NBLK - 3) // 3, ebody, 0)

    for b in (NBLK - 2, NBLK - 1):
        p = b % 3
        wait_idx(b, p)
        wait_gather(b, p)
        scale(p)
        scatter(p)
        wait_scatter((p + 2) % 3)
    wait_scatter((NBLK - 1) % 3)

    plsc.subcore_barrier()
    pltpu.sync_copy(agg_sh.at[pl.ds(s * CH, CH)],
                    out_hbm.at[c, pl.ds(s * CH, CH)])


RB = 1280


def _dinv_body(degp_ref, x_ref, w1_ref, table_ref, dinvb_ref):
    d = degp_ref[0] + degp_ref[1] + 1.0
    dv = lax.rsqrt(d)
    xw1 = jnp.dot(x_ref[...], w1_ref[...], preferred_element_type=jnp.float32)
    dt = dv.T
    for r in range(NR):
        blk = jnp.broadcast_to(dt[:, r:r + 1], (128, HD))
        sl = pl.ds(r * 128, 128)
        dinvb_ref[sl, :] = blk
        table_ref[sl, :] = blk * xw1[r * 128:(r + 1) * 128, :]


_dinv_tc = pl.pallas_call(
    _dinv_body,
    in_specs=[pl.BlockSpec((NC, NR, 128), lambda: (0, 0, 0)),
              pl.BlockSpec((NP, XD), lambda: (0, 0)),
              pl.BlockSpec((XD, HD), lambda: (0, 0))],
    out_specs=[pl.BlockSpec((NP, HD), lambda: (0, 0)),
               pl.BlockSpec((NP, HD), lambda: (0, 0))],
    out_shape=[jax.ShapeDtypeStruct((NP, HD), jnp.float32),
               jax.ShapeDtypeStruct((NP, HD), jnp.float32)],
    grid=(),
)


def _layer2_body(aggp_ref, t1_ref, dinvb_ref, b1_ref, w2_ref, o_ref):
    h = jax.nn.relu(dinvb_ref[...]
                    * (aggp_ref[0] + aggp_ref[1] + t1_ref[...])
                    + b1_ref[...])
    o_ref[...] = dinvb_ref[...] * jnp.dot(h, w2_ref[...],
                                          preferred_element_type=jnp.float32)


_layer2 = pl.pallas_call(
    _layer2_body,
    grid=(NP // RB,),
    in_specs=[pl.BlockSpec((NC, RB, HD), lambda i: (0, i, 0)),
              pl.BlockSpec((RB, HD), lambda i: (i, 0)),
              pl.BlockSpec((RB, HD), lambda i: (i, 0)),
              pl.BlockSpec((1, HD), lambda i: (0, 0)),
              pl.BlockSpec((HD, HD), lambda i: (0, 0))],
    out_specs=pl.BlockSpec((RB, HD), lambda i: (i, 0)),
    out_shape=jax.ShapeDtypeStruct((NP, HD), jnp.float32),
)


def _gru_body(aggp_ref, t2_ref, dinvb_ref, b2_ref, wz_ref, uz_ref, bz_ref,
              wr_ref, ur_ref, br_ref, wh_ref, uh_ref, bh_ref, wl_ref, bl_ref,
              out_ref, h_ref):
    def mm(a, b):
        return jnp.dot(a, b[...], preferred_element_type=jnp.float32)

    z = jnp.tanh(dinvb_ref[...] * (aggp_ref[0] + aggp_ref[1] + t2_ref[...])
                 + b2_ref[...])
    xz = mm(z, wz_ref) + bz_ref[...]
    xr = mm(z, wr_ref) + br_ref[...]
    xh = mm(z, wh_ref) + bh_ref[...]

    h1 = jax.nn.sigmoid(xz) * jnp.tanh(xh)
    zg = jax.nn.sigmoid(xz + mm(h1, uz_ref))
    rg = jax.nn.sigmoid(xr + mm(h1, ur_ref))
    hh = jnp.tanh(xh + mm(rg * h1, uh_ref))
    h2 = (1.0 - zg) * h1 + zg * hh
    zg = jax.nn.sigmoid(xz + mm(h2, uz_ref))
    rg = jax.nn.sigmoid(xr + mm(h2, ur_ref))
    hh = jnp.tanh(xh + mm(rg * h2, uh_ref))
    h3 = (1.0 - zg) * h2 + zg * hh

    bl = bl_ref[...]
    out_ref[0] = mm(h1, wl_ref) + bl
    out_ref[1] = mm(h2, wl_ref) + bl
    out_ref[2] = mm(h3, wl_ref) + bl
    h_ref[...] = h3


RB2 = 1000

_gru_tc = pl.pallas_call(
    _gru_body,
    grid=(N // RB2,),
    in_specs=[pl.BlockSpec((NC, RB2, HD), lambda i: (0, i, 0)),
              pl.BlockSpec((RB2, HD), lambda i: (i, 0)),
              pl.BlockSpec((RB2, HD), lambda i: (i, 0)),
              pl.BlockSpec((1, HD), lambda i: (0, 0)),
              pl.BlockSpec((HD, HD), lambda i: (0, 0)),
              pl.BlockSpec((HD, HD), lambda i: (0, 0)),
              pl.BlockSpec((1, HD), lambda i: (0, 0)),
              pl.BlockSpec((HD, HD), lambda i: (0, 0)),
              pl.BlockSpec((HD, HD), lambda i: (0, 0)),
              pl.BlockSpec((1, HD), lambda i: (0, 0)),
              pl.BlockSpec((HD, HD), lambda i: (0, 0)),
              pl.BlockSpec((HD, HD), lambda i: (0, 0)),
              pl.BlockSpec((1, HD), lambda i: (0, 0)),
              pl.BlockSpec((HD, ZD), lambda i: (0, 0)),
              pl.BlockSpec((1, ZD), lambda i: (0, 0))],
    out_specs=[pl.BlockSpec((T, RB2, ZD), lambda i: (0, i, 0)),
               pl.BlockSpec((RB2, HD), lambda i: (i, 0))],
    out_shape=[jax.ShapeDtypeStruct((T, N, ZD), jnp.float32),
               jax.ShapeDtypeStruct((N, HD), jnp.float32)],
)


def kernel(x, edge_index, edge_weight, W1, b1, W2, b2, Wz, Uz, bz,
           Wr, Ur, br, Wh, Uh, bh, Wl, bl):
    pad = EP - E
    fill = jnp.arange(pad, dtype=jnp.int32) % N
    src = jnp.concatenate([edge_index[0], fill])
    dst = jnp.concatenate([edge_index[1], fill])
    w = jnp.pad(edge_weight, (0, pad))
    src3 = src.reshape(NW, NBLK, EB)
    dst3 = dst.reshape(NW * NBLK, EB)
    w3 = w.reshape(NW * NBLK, EB)

    xp = jnp.pad(x, ((0, NP - N), (0, 0)))

    degp = _deg_kernel(dst, w)
    table1, dinvb = _dinv_tc(degp.reshape(NC, NR, 128), xp, W1)

    agg1 = _agg_kernel(table1, src3, dst3, w3)
    table2 = _layer2(agg1, table1, dinvb, b1.reshape(1, HD), W2)
    agg2 = _agg_kernel(table2, src3, dst3, w3)

    out, h = _gru_tc(agg2, table2, dinvb, b2.reshape(1, HD), Wz, Uz,
                     bz.reshape(1, HD), Wr, Ur, br.reshape(1, HD),
                     Wh, Uh, bh.reshape(1, HD), Wl, bl.reshape(1, ZD))
    return out, h
